# baseline (speedup 1.0000x reference)
import jax
import jax.numpy as jnp
from jax.experimental import pallas as pl

_ALPHA = 1.0


def _blend_kernel(image_ref, label_ref, outlier_ref, mask_ref,
                  image_out_ref, label_out_ref):
    m = mask_ref[...]
    img = image_ref[...]
    out = outlier_ref[...]
    blended = img - _ALPHA * m[:, None] * img + _ALPHA * out
    image_out_ref[...] = blended.astype(jnp.uint8)
    lbl = label_ref[...]
    label_out_ref[...] = jnp.where(m != 0.0, jnp.int32(100), lbl)


def kernel(image, label, outlier, mask):
    N, C, H, W = image.shape
    HB = 256
    grid = (N, H // HB)

    img_spec = pl.BlockSpec((1, C, HB, W), lambda n, h: (n, 0, h, 0))
    map_spec = pl.BlockSpec((1, HB, W), lambda n, h: (n, h, 0))

    image_out, label_out = pl.pallas_call(
        _blend_kernel,
        grid=grid,
        in_specs=[img_spec, map_spec, img_spec, map_spec],
        out_specs=[img_spec, map_spec],
        out_shape=[
            jax.ShapeDtypeStruct((N, C, H, W), jnp.uint8),
            jax.ShapeDtypeStruct((N, H, W), label.dtype),
        ],
    )(image, label, outlier, mask)
    return (image_out, label_out)


# speedup vs baseline: 1.0326x; 1.0326x over previous
import jax
import jax.numpy as jnp
from jax.experimental import pallas as pl

_ALPHA = 1.0


def _blend_kernel(image_ref, label_ref, outlier_ref, mask_ref,
                  image_out_ref, label_out_ref):
    m = mask_ref[...]
    img = image_ref[...]
    out = outlier_ref[...]
    blended = img - _ALPHA * m[:, None] * img + _ALPHA * out
    image_out_ref[...] = blended.astype(jnp.uint8)
    lbl = label_ref[...]
    label_out_ref[...] = jnp.where(m != 0.0, jnp.int32(100), lbl)


def kernel(image, label, outlier, mask):
    N, C, H, W = image.shape
    NB = 2
    HB = H
    grid = (N // NB,)

    img_spec = pl.BlockSpec((NB, C, HB, W), lambda n: (n, 0, 0, 0))
    map_spec = pl.BlockSpec((NB, HB, W), lambda n: (n, 0, 0))

    image_out, label_out = pl.pallas_call(
        _blend_kernel,
        grid=grid,
        in_specs=[img_spec, map_spec, img_spec, map_spec],
        out_specs=[img_spec, map_spec],
        out_shape=[
            jax.ShapeDtypeStruct((N, C, H, W), jnp.uint8),
            jax.ShapeDtypeStruct((N, H, W), label.dtype),
        ],
    )(image, label, outlier, mask)
    return (image_out, label_out)


# speedup vs baseline: 1.0504x; 1.0173x over previous
import jax
import jax.numpy as jnp
from jax.experimental import pallas as pl

_ALPHA = 1.0


def _blend_kernel(image_ref, label_ref, outlier_ref, mask_ref,
                  image_out_ref, label_out_ref):
    m = mask_ref[...]
    img = image_ref[...]
    out = outlier_ref[...]
    blended = img - _ALPHA * m[:, None] * img + _ALPHA * out
    image_out_ref[...] = blended.astype(jnp.uint8)
    lbl = label_ref[...]
    label_out_ref[...] = jnp.where(m != 0.0, jnp.int32(100), lbl)


def kernel(image, label, outlier, mask):
    N, C, H, W = image.shape
    NB = 1
    HB = H
    grid = (N // NB,)

    img_spec = pl.BlockSpec((NB, C, HB, W), lambda n: (n, 0, 0, 0))
    map_spec = pl.BlockSpec((NB, HB, W), lambda n: (n, 0, 0))

    image_out, label_out = pl.pallas_call(
        _blend_kernel,
        grid=grid,
        in_specs=[img_spec, map_spec, img_spec, map_spec],
        out_specs=[img_spec, map_spec],
        out_shape=[
            jax.ShapeDtypeStruct((N, C, H, W), jnp.uint8),
            jax.ShapeDtypeStruct((N, H, W), label.dtype),
        ],
    )(image, label, outlier, mask)
    return (image_out, label_out)
